# Initial kernel scaffold; baseline (speedup 1.0000x reference)
import functools

import jax
import jax.numpy as jnp
from jax import lax
from jax.experimental import pallas as pl
from jax.experimental.pallas import tpu as pltpu
from jax.experimental.pallas import tpu_sc as plsc

_INFO = plsc.get_sparse_core_info()
_NC = _INFO.num_cores
_NS = _INFO.num_subcores
_NW = _NC * _NS

_CHUNK = 128


def _make_sc_call(n_total, emb_w, emb_f, n_chunks):
    d_out = emb_w + emb_f
    n_per_w = n_chunks * _CHUNK
    mesh = plsc.VectorSubcoreMesh(core_axis_name="c", subcore_axis_name="s")

    @functools.partial(
        pl.kernel,
        out_type=jax.ShapeDtypeStruct((n_total, d_out), jnp.float32),
        mesh=mesh,
        scratch_types=[
            pltpu.VMEM((n_chunks, _CHUNK), jnp.int32),
            pltpu.VMEM((n_chunks, _CHUNK), jnp.int32),
            pltpu.VMEM((_CHUNK, emb_w), jnp.float32),
            pltpu.VMEM((_CHUNK, emb_f), jnp.float32),
            pltpu.SemaphoreType.DMA,
            pltpu.SemaphoreType.DMA,
        ],
    )
    def sc_call(idx_w_hbm, idx_f_hbm, wtab_hbm, ftab_hbm, out_hbm,
                idx_w_v, idx_f_v, wrows_v, frows_v, sem_w, sem_f):
        wid = lax.axis_index("s") * _NC + lax.axis_index("c")
        base = wid * n_per_w
        pltpu.sync_copy(idx_w_hbm.at[wid], idx_w_v)
        pltpu.sync_copy(idx_f_hbm.at[wid], idx_f_v)

        def body(g, carry):
            row0 = base + g * _CHUNK
            cp_w = pltpu.async_copy(wtab_hbm.at[idx_w_v.at[g]], wrows_v, sem_w)
            cp_f = pltpu.async_copy(ftab_hbm.at[idx_f_v.at[g]], frows_v, sem_f)
            cp_w.wait()
            cp_f.wait()
            pltpu.sync_copy(
                wrows_v, out_hbm.at[pl.ds(row0, _CHUNK), pl.ds(0, emb_w)])
            pltpu.sync_copy(
                frows_v, out_hbm.at[pl.ds(row0, _CHUNK), pl.ds(emb_w, emb_f)])
            return carry

        lax.fori_loop(0, n_chunks, body, 0)

    return sc_call


def kernel(word_inputs, feature_inputs, word_seq_lengths, word_table, feat_table):
    del word_seq_lengths
    b, l = word_inputs.shape
    vocab, emb_w = word_table.shape
    _, emb_f = feat_table.shape
    n_total = b * l
    assert n_total % (_NW * _CHUNK) == 0
    n_chunks = n_total // (_NW * _CHUNK)

    idx_w = word_inputs.astype(jnp.int32).reshape(_NW, n_chunks, _CHUNK)
    idx_f = feature_inputs[0].astype(jnp.int32).reshape(_NW, n_chunks, _CHUNK)

    sc_call = _make_sc_call(n_total, emb_w, emb_f, n_chunks)
    out = sc_call(idx_w, idx_f, word_table, feat_table)
    return out.reshape(b, l, emb_w + emb_f)


# speedup vs baseline: 4.5886x; 4.5886x over previous
import functools

import jax
import jax.numpy as jnp
from jax import lax
from jax.experimental import pallas as pl
from jax.experimental.pallas import tpu as pltpu
from jax.experimental.pallas import tpu_sc as plsc

_INFO = plsc.get_sparse_core_info()
_NC = _INFO.num_cores
_NS = _INFO.num_subcores
_NW = _NC * _NS

_CHUNK = 128


def _make_sc_call(n_total, emb_w, emb_f, n_chunks):
    d_out = emb_w + emb_f
    n_per_w = n_chunks * _CHUNK
    mesh = plsc.VectorSubcoreMesh(core_axis_name="c", subcore_axis_name="s")

    @functools.partial(
        pl.kernel,
        out_type=jax.ShapeDtypeStruct((n_total, d_out), jnp.float32),
        mesh=mesh,
        scratch_types=[
            pltpu.VMEM((n_chunks, _CHUNK), jnp.int32),
            pltpu.VMEM((n_chunks, _CHUNK), jnp.int32),
            pltpu.VMEM((_CHUNK, emb_w), jnp.float32),
            pltpu.VMEM((_CHUNK, emb_f), jnp.float32),
            pltpu.SemaphoreType.DMA,
            pltpu.SemaphoreType.DMA,
        ],
        compiler_params=pltpu.CompilerParams(use_tc_tiling_on_sc=False),
    )
    def sc_call(idx_w_hbm, idx_f_hbm, wtab_hbm, ftab_hbm, out_hbm,
                idx_w_v, idx_f_v, wrows_v, frows_v, sem_w, sem_f):
        wid = lax.axis_index("s") * _NC + lax.axis_index("c")
        base = wid * n_per_w
        pltpu.sync_copy(idx_w_hbm.at[wid], idx_w_v)
        pltpu.sync_copy(idx_f_hbm.at[wid], idx_f_v)

        def body(g, carry):
            row0 = base + g * _CHUNK
            cp_w = pltpu.async_copy(wtab_hbm.at[idx_w_v.at[g]], wrows_v, sem_w)
            cp_f = pltpu.async_copy(ftab_hbm.at[idx_f_v.at[g]], frows_v, sem_f)
            cp_w.wait()
            cp_f.wait()
            pltpu.sync_copy(
                wrows_v, out_hbm.at[pl.ds(row0, _CHUNK), pl.ds(0, emb_w)])
            pltpu.sync_copy(
                frows_v, out_hbm.at[pl.ds(row0, _CHUNK), pl.ds(emb_w, emb_f)])
            return carry

        lax.fori_loop(0, n_chunks, body, 0)

    return sc_call


def kernel(word_inputs, feature_inputs, word_seq_lengths, word_table, feat_table):
    del word_seq_lengths
    b, l = word_inputs.shape
    vocab, emb_w = word_table.shape
    _, emb_f = feat_table.shape
    n_total = b * l
    assert n_total % (_NW * _CHUNK) == 0
    n_chunks = n_total // (_NW * _CHUNK)

    idx_w = word_inputs.astype(jnp.int32).reshape(_NW, n_chunks, _CHUNK)
    idx_f = feature_inputs[0].astype(jnp.int32).reshape(_NW, n_chunks, _CHUNK)

    sc_call = _make_sc_call(n_total, emb_w, emb_f, n_chunks)
    out = sc_call(idx_w, idx_f, word_table, feat_table)
    return out.reshape(b, l, emb_w + emb_f)


# speedup vs baseline: 4.6514x; 1.0137x over previous
import functools

import jax
import jax.numpy as jnp
from jax import lax
from jax.experimental import pallas as pl
from jax.experimental.pallas import tpu as pltpu
from jax.experimental.pallas import tpu_sc as plsc

_INFO = plsc.get_sparse_core_info()
_NC = _INFO.num_cores
_NS = _INFO.num_subcores
_NW = _NC * _NS

_CHUNK = 128
_NBUF = 4


def _make_sc_call(n_total, emb_w, emb_f, n_chunks):
    d_out = emb_w + emb_f
    n_per_w = n_chunks * _CHUNK
    assert n_chunks % _NBUF == 0 and n_chunks // _NBUF >= 2
    mesh = plsc.VectorSubcoreMesh(core_axis_name="c", subcore_axis_name="s")

    scratch = [
        pltpu.VMEM((n_chunks, _CHUNK), jnp.int32),
        pltpu.VMEM((n_chunks, _CHUNK), jnp.int32),
    ]
    scratch += [pltpu.VMEM((_CHUNK, emb_w), jnp.float32) for _ in range(_NBUF)]
    scratch += [pltpu.VMEM((_CHUNK, emb_f), jnp.float32) for _ in range(_NBUF)]
    scratch += [pltpu.SemaphoreType.DMA for _ in range(2 * _NBUF)]

    @functools.partial(
        pl.kernel,
        out_type=jax.ShapeDtypeStruct((n_total, d_out), jnp.float32),
        mesh=mesh,
        scratch_types=scratch,
        compiler_params=pltpu.CompilerParams(use_tc_tiling_on_sc=False),
    )
    def sc_call(idx_w_hbm, idx_f_hbm, wtab_hbm, ftab_hbm, out_hbm, *refs):
        idx_w_v, idx_f_v = refs[0], refs[1]
        wbufs = refs[2:2 + _NBUF]
        fbufs = refs[2 + _NBUF:2 + 2 * _NBUF]
        gsems = refs[2 + 2 * _NBUF:2 + 3 * _NBUF]
        wsems = refs[2 + 3 * _NBUF:2 + 4 * _NBUF]

        wid = lax.axis_index("s") * _NC + lax.axis_index("c")
        base = wid * n_per_w
        pltpu.sync_copy(idx_w_hbm.at[wid], idx_w_v)
        pltpu.sync_copy(idx_f_hbm.at[wid], idx_f_v)

        def start_gathers(g, b):
            pltpu.async_copy(wtab_hbm.at[idx_w_v.at[g]], wbufs[b], gsems[b])
            pltpu.async_copy(ftab_hbm.at[idx_f_v.at[g]], fbufs[b], gsems[b])

        def wait_gathers(b):
            pltpu.make_async_copy(
                wtab_hbm.at[idx_w_v.at[0]], wbufs[b], gsems[b]).wait()
            pltpu.make_async_copy(
                ftab_hbm.at[idx_f_v.at[0]], fbufs[b], gsems[b]).wait()

        def out_w(g):
            return out_hbm.at[pl.ds(base + g * _CHUNK, _CHUNK), pl.ds(0, emb_w)]

        def out_f(g):
            return out_hbm.at[
                pl.ds(base + g * _CHUNK, _CHUNK), pl.ds(emb_w, emb_f)]

        def start_writes(g, b):
            pltpu.async_copy(wbufs[b], out_w(g), wsems[b])
            pltpu.async_copy(fbufs[b], out_f(g), wsems[b])

        def wait_writes(g, b):
            pltpu.make_async_copy(wbufs[b], out_w(g), wsems[b]).wait()
            pltpu.make_async_copy(fbufs[b], out_f(g), wsems[b]).wait()

        for b in range(_NBUF):
            start_gathers(b, b)

        def body(blk, carry):
            g0 = blk * _NBUF
            for b in range(_NBUF):
                g = g0 + b
                wait_gathers(b)
                start_writes(g, b)
                wait_writes(g, b)
                start_gathers(g + _NBUF, b)
            return carry

        lax.fori_loop(0, n_chunks // _NBUF - 1, body, 0)

        g0 = n_chunks - _NBUF
        for b in range(_NBUF):
            wait_gathers(b)
            start_writes(g0 + b, b)
        for b in range(_NBUF):
            wait_writes(g0 + b, b)

    return sc_call


def kernel(word_inputs, feature_inputs, word_seq_lengths, word_table, feat_table):
    del word_seq_lengths
    b, l = word_inputs.shape
    vocab, emb_w = word_table.shape
    _, emb_f = feat_table.shape
    n_total = b * l
    assert n_total % (_NW * _CHUNK) == 0
    n_chunks = n_total // (_NW * _CHUNK)

    idx_w = word_inputs.astype(jnp.int32).reshape(_NW, n_chunks, _CHUNK)
    idx_f = feature_inputs[0].astype(jnp.int32).reshape(_NW, n_chunks, _CHUNK)

    sc_call = _make_sc_call(n_total, emb_w, emb_f, n_chunks)
    out = sc_call(idx_w, idx_f, word_table, feat_table)
    return out.reshape(b, l, emb_w + emb_f)
